# baseline (speedup 1.0000x reference)
import jax
import jax.numpy as jnp
from jax.experimental import pallas as pl


def _update_kernel(idx_ref, x_ref, w_ref, b_ref, m_ref, k_ref, v_ref, o_ref):
    n = pl.program_id(0)
    w = w_ref[0]
    logits = jax.lax.dot_general(
        x_ref[...], w, (((1,), (1,)), ((), ())),
        preferred_element_type=jnp.float32)
    logits = logits + b_ref[0, 0][None, :]
    alpha = jax.nn.sigmoid(logits)
    active = jnp.any(idx_ref[...] == n, axis=1)
    m = m_ref[:, 0]
    k = k_ref[:, 0, 0]
    v = v_ref[:, 0, 0]
    kv = k[:, :, None] * v[:, None, :]
    upd = m * alpha[:, :, None] + kv
    o_ref[:, 0] = jnp.where(active[:, None, None], upd, m)


def kernel(M, M_k, M_v, indices_update, x_t, W, b):
    B, N, D, _ = M.shape
    input_dim = x_t.shape[1]
    idx = indices_update.astype(jnp.int32)
    W3 = W.reshape(N, D, input_dim)
    b3 = b.reshape(N, 1, D)
    Mk4 = M_k.reshape(B, N, 1, D)
    Mv4 = M_v.reshape(B, N, 1, D)

    return pl.pallas_call(
        _update_kernel,
        grid=(N,),
        in_specs=[
            pl.BlockSpec(idx.shape, lambda n: (0, 0)),
            pl.BlockSpec((B, input_dim), lambda n: (0, 0)),
            pl.BlockSpec((1, D, input_dim), lambda n: (n, 0, 0)),
            pl.BlockSpec((1, 1, D), lambda n: (n, 0, 0)),
            pl.BlockSpec((B, 1, D, D), lambda n: (0, n, 0, 0)),
            pl.BlockSpec((B, 1, 1, D), lambda n: (0, n, 0, 0)),
            pl.BlockSpec((B, 1, 1, D), lambda n: (0, n, 0, 0)),
        ],
        out_specs=pl.BlockSpec((B, 1, D, D), lambda n: (0, n, 0, 0)),
        out_shape=jax.ShapeDtypeStruct((B, N, D, D), M.dtype),
    )(idx, x_t, W3, b3, M, Mk4, Mv4)
